# baseline (speedup 1.0000x reference)
import functools

import jax
import jax.numpy as jnp
from jax import lax
from jax.experimental import pallas as pl
from jax.experimental.pallas import tpu as pltpu
from jax.experimental.pallas import tpu_sc as plsc

_ROWS = 128
_COLS = 32768
_LANES = 16
_VREGS_PER_ROW = _COLS // _LANES
_NUM_WORKERS = 32
_ROWS_PER_WORKER = _ROWS // _NUM_WORKERS
_REGION = 512
_CAND_BUF = _REGION * _LANES
_NEWTON_ITERS = 12
_UNROLL = 16
_FLT_UNROLL = 8
_SENTINEL = -1.0e30


def _sc_process_row(row_v, cand_v, stat_m, stat_t, r):
    def max_body(i, carry):
        a0, a1 = carry
        base = i * _UNROLL
        for u in range(0, _UNROLL, 2):
            a0 = jnp.maximum(a0, row_v[pl.ds((base + u) * _LANES, _LANES)])
            a1 = jnp.maximum(a1, row_v[pl.ds((base + u + 1) * _LANES, _LANES)])
        return a0, a1

    neg = jnp.full((_LANES,), -3.0e38, jnp.float32)
    a0, a1 = lax.fori_loop(0, _VREGS_PER_ROW // _UNROLL, max_body, (neg, neg))
    m = jnp.max(jnp.maximum(a0, a1))
    thr_v = jnp.full((_LANES,), m - 2.0, jnp.float32)

    def flt_body(i, off):
        base = i * _FLT_UNROLL
        vs = [row_v[pl.ds((base + u) * _LANES, _LANES)]
              for u in range(_FLT_UNROLL)]
        msks = [v >= thr_v for v in vs]
        pcs = [plsc.all_reduce_population_count(mk) for mk in msks]
        for u in range(_FLT_UNROLL):
            plsc.store_compressed(cand_v.at[pl.ds(off, _LANES)], vs[u],
                                  mask=msks[u])
            off = jnp.minimum(off + pcs[u][0], _CAND_BUF - _LANES)
        return off

    n_cand = lax.fori_loop(0, _VREGS_PER_ROW // _FLT_UNROLL, flt_body,
                           jnp.int32(0))
    cand_v[pl.ds(jnp.minimum(n_cand, _CAND_BUF - _LANES), _LANES)] = jnp.full(
        (_LANES,), _SENTINEL, jnp.float32)
    n_vregs = (n_cand + _LANES - 1) >> 4

    m_v = jnp.full((_LANES,), m, jnp.float32)

    def nrm_body(i, carry):
        c = cand_v[pl.ds(i * _LANES, _LANES)]
        cand_v[pl.ds(i * _LANES, _LANES)] = (c - m_v) * 0.5
        return carry

    lax.fori_loop(0, n_vregs, nrm_body, 0)

    def newton_body(kk, tau_v):
        def acc_body(i, carry):
            fa, sa = carry
            c = cand_v[pl.ds(i * _LANES, _LANES)]
            p = jnp.maximum(c - tau_v, 0.0)
            return fa + p * p, sa + p

        z = jnp.zeros((_LANES,), jnp.float32)
        fa, sa = lax.fori_loop(0, n_vregs, acc_body, (z, z))
        f_v = jnp.full((_LANES,), jnp.sum(fa), jnp.float32)
        s_v = jnp.full((_LANES,), jnp.sum(sa), jnp.float32)
        return tau_v + (f_v - 1.0) / jnp.maximum(2.0 * s_v, 1e-30)

    tau_v = lax.fori_loop(0, _NEWTON_ITERS, newton_body,
                          jnp.full((_LANES,), -1.0, jnp.float32))

    stat_m[r] = m_v
    stat_t[r] = tau_v


def _sc_stats_kernel(rpw, x_hbm, maxs_hbm, taus_hbm,
                     row0_v, row1_v, cand_v, stat_m, stat_t, sem0, sem1):
    wid = lax.axis_index("s") * 2 + lax.axis_index("c")
    base_row = wid * rpw
    bufs = (row0_v, row1_v)
    sems = (sem0, sem1)

    pltpu.async_copy(x_hbm.at[base_row], bufs[0], sems[0])
    for r in range(rpw):
        b = r % 2
        if r + 1 < rpw:
            pltpu.async_copy(x_hbm.at[base_row + r + 1],
                             bufs[1 - b], sems[1 - b])
        pltpu.make_async_copy(x_hbm.at[base_row + r], bufs[b],
                              sems[b]).wait()
        _sc_process_row(bufs[b], cand_v, stat_m, stat_t, r)

    pltpu.sync_copy(stat_m, maxs_hbm.at[pl.ds(base_row, rpw)])
    pltpu.sync_copy(stat_t, taus_hbm.at[pl.ds(base_row, rpw)])


def _sc_stats(x):
    rows = x.shape[0]
    rpw = rows // _NUM_WORKERS
    mesh = plsc.VectorSubcoreMesh(core_axis_name="c", subcore_axis_name="s")
    kfn = functools.partial(
        pl.kernel,
        out_type=[
            jax.ShapeDtypeStruct((rows, _LANES), jnp.float32),
            jax.ShapeDtypeStruct((rows, _LANES), jnp.float32),
        ],
        mesh=mesh,
        scratch_types=[
            pltpu.VMEM((_COLS,), jnp.float32),
            pltpu.VMEM((_COLS,), jnp.float32),
            pltpu.VMEM((_CAND_BUF,), jnp.float32),
            pltpu.VMEM((rpw, _LANES), jnp.float32),
            pltpu.VMEM((rpw, _LANES), jnp.float32),
            pltpu.SemaphoreType.DMA,
            pltpu.SemaphoreType.DMA,
        ],
        compiler_params=pltpu.CompilerParams(needs_layout_passes=False),
    )(functools.partial(_sc_stats_kernel, rpw))
    return kfn(x)


_TC_BLOCK_ROWS = 16


def _tc_finish_block(x_ref, maxs_ref, taus_ref, o_ref):
    m = maxs_ref[:, 0:1]
    tau = taus_ref[:, 0:1]
    t = jnp.maximum((x_ref[...] - m) * 0.5 - tau, 0.0)
    o_ref[...] = t * t


def kernel(logits):
    maxs, taus = _sc_stats(logits)
    grid = (_ROWS // _TC_BLOCK_ROWS,)
    return pl.pallas_call(
        _tc_finish_block,
        grid=grid,
        in_specs=[
            pl.BlockSpec((_TC_BLOCK_ROWS, _COLS), lambda i: (i, 0)),
            pl.BlockSpec((_TC_BLOCK_ROWS, _LANES), lambda i: (i, 0)),
            pl.BlockSpec((_TC_BLOCK_ROWS, _LANES), lambda i: (i, 0)),
        ],
        out_specs=pl.BlockSpec((_TC_BLOCK_ROWS, _COLS), lambda i: (i, 0)),
        out_shape=jax.ShapeDtypeStruct((_ROWS, _COLS), logits.dtype),
    )(logits, maxs, taus)


# speedup vs baseline: 1.0607x; 1.0607x over previous
import functools

import jax
import jax.numpy as jnp
from jax import lax
from jax.experimental import pallas as pl
from jax.experimental.pallas import tpu as pltpu
from jax.experimental.pallas import tpu_sc as plsc

_ROWS = 128
_COLS = 32768
_LANES = 16
_VREGS_PER_ROW = _COLS // _LANES
_NUM_WORKERS = 32
_ROWS_PER_WORKER = _ROWS // _NUM_WORKERS
_REGION = 512
_CAND_BUF = _REGION * _LANES
_NEWTON_ITERS = 12
_UNROLL = 16
_SENTINEL = -1.0e30


def _sc_process_row(row_v, cand_v, stat_m, stat_t, r):
    def max_body(i, carry):
        a0, a1 = carry
        base = i * _UNROLL
        for u in range(0, _UNROLL, 2):
            a0 = jnp.maximum(a0, row_v[pl.ds((base + u) * _LANES, _LANES)])
            a1 = jnp.maximum(a1, row_v[pl.ds((base + u + 1) * _LANES, _LANES)])
        return a0, a1

    neg = jnp.full((_LANES,), -3.0e38, jnp.float32)
    a0, a1 = lax.fori_loop(0, _VREGS_PER_ROW // _UNROLL, max_body, (neg, neg))
    m = jnp.max(jnp.maximum(a0, a1))
    thr_v = jnp.full((_LANES,), m - 2.0, jnp.float32)

    def flt_body(i, off_v):
        base = i * _UNROLL
        vs = [row_v[pl.ds((base + u) * _LANES, _LANES)] for u in range(_UNROLL)]
        msks = [v >= thr_v for v in vs]
        pcs = [plsc.all_reduce_population_count(mk) for mk in msks]
        pre = off_v
        for u in range(_UNROLL):
            pos = pre + plsc.cumsum(msks[u].astype(jnp.int32)) - 1
            pos = jnp.minimum(pos, _CAND_BUF - 1)
            plsc.store_scatter(cand_v, [pos], vs[u], mask=msks[u])
            pre = pre + pcs[u]
        return pre

    off_v = lax.fori_loop(0, _VREGS_PER_ROW // _UNROLL, flt_body,
                          jnp.zeros((_LANES,), jnp.int32))
    n_cand = off_v[0]
    cand_v[pl.ds(jnp.minimum(n_cand, _CAND_BUF - _LANES), _LANES)] = jnp.full(
        (_LANES,), _SENTINEL, jnp.float32)
    n_vregs = (n_cand + _LANES - 1) >> 4

    m_v = jnp.full((_LANES,), m, jnp.float32)

    def nrm_body(i, carry):
        c = cand_v[pl.ds(i * _LANES, _LANES)]
        cand_v[pl.ds(i * _LANES, _LANES)] = (c - m_v) * 0.5
        return carry

    lax.fori_loop(0, n_vregs, nrm_body, 0)

    def newton_body(kk, tau_v):
        def acc_body(i, carry):
            fa, sa = carry
            c = cand_v[pl.ds(i * _LANES, _LANES)]
            p = jnp.maximum(c - tau_v, 0.0)
            return fa + p * p, sa + p

        z = jnp.zeros((_LANES,), jnp.float32)
        fa, sa = lax.fori_loop(0, n_vregs, acc_body, (z, z))
        f_v = jnp.full((_LANES,), jnp.sum(fa), jnp.float32)
        s_v = jnp.full((_LANES,), jnp.sum(sa), jnp.float32)
        return tau_v + (f_v - 1.0) / jnp.maximum(2.0 * s_v, 1e-30)

    tau_v = lax.fori_loop(0, _NEWTON_ITERS, newton_body,
                          jnp.full((_LANES,), -1.0, jnp.float32))

    stat_m[r] = m_v
    stat_t[r] = tau_v


def _sc_stats_kernel(rpw, x_hbm, maxs_hbm, taus_hbm,
                     row0_v, row1_v, cand_v, stat_m, stat_t, sem0, sem1):
    wid = lax.axis_index("s") * 2 + lax.axis_index("c")
    base_row = wid * rpw
    bufs = (row0_v, row1_v)
    sems = (sem0, sem1)

    pltpu.async_copy(x_hbm.at[base_row], bufs[0], sems[0])
    for r in range(rpw):
        b = r % 2
        if r + 1 < rpw:
            pltpu.async_copy(x_hbm.at[base_row + r + 1],
                             bufs[1 - b], sems[1 - b])
        pltpu.make_async_copy(x_hbm.at[base_row + r], bufs[b],
                              sems[b]).wait()
        _sc_process_row(bufs[b], cand_v, stat_m, stat_t, r)

    pltpu.sync_copy(stat_m, maxs_hbm.at[pl.ds(base_row, rpw)])
    pltpu.sync_copy(stat_t, taus_hbm.at[pl.ds(base_row, rpw)])


def _sc_stats(x):
    rows = x.shape[0]
    rpw = rows // _NUM_WORKERS
    mesh = plsc.VectorSubcoreMesh(core_axis_name="c", subcore_axis_name="s")
    kfn = functools.partial(
        pl.kernel,
        out_type=[
            jax.ShapeDtypeStruct((rows, _LANES), jnp.float32),
            jax.ShapeDtypeStruct((rows, _LANES), jnp.float32),
        ],
        mesh=mesh,
        scratch_types=[
            pltpu.VMEM((_COLS,), jnp.float32),
            pltpu.VMEM((_COLS,), jnp.float32),
            pltpu.VMEM((_CAND_BUF,), jnp.float32),
            pltpu.VMEM((rpw, _LANES), jnp.float32),
            pltpu.VMEM((rpw, _LANES), jnp.float32),
            pltpu.SemaphoreType.DMA,
            pltpu.SemaphoreType.DMA,
        ],
        compiler_params=pltpu.CompilerParams(needs_layout_passes=False),
    )(functools.partial(_sc_stats_kernel, rpw))
    return kfn(x)


_TC_BLOCK_ROWS = 16


def _tc_finish_block(x_ref, maxs_ref, taus_ref, o_ref):
    m = maxs_ref[:, 0:1]
    tau = taus_ref[:, 0:1]
    t = jnp.maximum((x_ref[...] - m) * 0.5 - tau, 0.0)
    o_ref[...] = t * t


def kernel(logits):
    maxs, taus = _sc_stats(logits)
    grid = (_ROWS // _TC_BLOCK_ROWS,)
    return pl.pallas_call(
        _tc_finish_block,
        grid=grid,
        in_specs=[
            pl.BlockSpec((_TC_BLOCK_ROWS, _COLS), lambda i: (i, 0)),
            pl.BlockSpec((_TC_BLOCK_ROWS, _LANES), lambda i: (i, 0)),
            pl.BlockSpec((_TC_BLOCK_ROWS, _LANES), lambda i: (i, 0)),
        ],
        out_specs=pl.BlockSpec((_TC_BLOCK_ROWS, _COLS), lambda i: (i, 0)),
        out_shape=jax.ShapeDtypeStruct((_ROWS, _COLS), logits.dtype),
    )(logits, maxs, taus)
